# baseline (speedup 1.0000x reference)
import jax
import jax.numpy as jnp
from jax import lax
from jax.experimental import pallas as pl
from jax.experimental.pallas import tpu as pltpu
from jax.experimental.pallas import tpu_sc as plsc

N_NODES = 100000
MEM_DIM = 128
BATCH = 16384

L = 16
NT = 32
SZ = N_NODES // NT
W_PAD = ((SZ + L - 1) // L) * L
CH = 64


def _sc_body(idx_hbm, incret_hbm, inc_hbm, sqr_hbm, out_ref,
             idx_buf, owned_pos, wtab, win_idx, win_pos,
             idxchunk, idxnchunk, poschunk, buf0, buf1, buf2,
             idxchunk2, idxnchunk2, poschunk2, buf0b, buf1b, buf2b,
             sem_g, sem_s, sem_g2, sem_s2):
    cid = lax.axis_index("c")
    sid = lax.axis_index("s")
    wid = sid * 2 + cid
    base = wid * SZ

    pltpu.sync_copy(idx_hbm, idx_buf)

    def _init(j, _):
        wtab[pl.ds(j * L, L)] = jnp.full((L,), -1, jnp.int32)
        return 0
    lax.fori_loop(0, W_PAD // L, _init, 0)

    def _pass_a(j, cnt):
        v = idx_buf[pl.ds(j * L, L)]
        m = (v >= base) & (v < base + SZ)
        mi = m.astype(jnp.int32)
        s = plsc.cumsum(mi)
        off = cnt + s - 1
        posv = j * L + lax.iota(jnp.int32, L)
        plsc.store_scatter(owned_pos, [off], posv, mask=m)
        return cnt + s[15]
    cnt = lax.fori_loop(0, BATCH // L, _pass_a, jnp.int32(0))

    def _pass_b(j, _):
        valid = j * L + lax.iota(jnp.int32, L) < cnt
        p = owned_pos[pl.ds(j * L, L)]
        v = plsc.load_gather(idx_buf, [p], mask=valid)
        local = v - base
        _, lastm = plsc.scan_count(local, mask=valid)
        plsc.store_scatter(wtab, [local], p, mask=lastm & valid)
        return 0
    lax.fori_loop(0, (cnt + L - 1) // L, _pass_b, 0)

    def _pass_c(j, wcnt):
        w = wtab[pl.ds(j * L, L)]
        m = w >= 0
        mi = m.astype(jnp.int32)
        s = plsc.cumsum(mi)
        off = wcnt + s - 1
        nodev = base + j * L + lax.iota(jnp.int32, L)
        plsc.store_scatter(win_idx, [off], nodev, mask=m)
        plsc.store_scatter(win_pos, [off], w, mask=m)
        return wcnt + s[15]
    wcnt = lax.fori_loop(0, W_PAD // L, _pass_c, jnp.int32(0))

    nch = (wcnt + CH - 1) // CH

    @pl.when(wcnt > 0)
    def _pad():
        def _fill(j, _):
            inds = wcnt + j * L + lax.iota(jnp.int32, L)
            m = inds < nch * CH
            srcs = (inds - wcnt) % wcnt
            nv = plsc.load_gather(win_idx, [srcs], mask=m)
            pv = plsc.load_gather(win_pos, [srcs], mask=m)
            plsc.store_scatter(win_idx, [inds], nv, mask=m)
            plsc.store_scatter(win_pos, [inds], pv, mask=m)
            return 0
        lax.fori_loop(0, CH // L, _fill, 0)

    def _stage(ci, idxc, idxnc, posc):
        cb = ci * CH

        def _st(k, _):
            v = win_idx[pl.ds(cb + k * L, L)]
            idxc[pl.ds(k * L, L)] = v
            idxnc[pl.ds(k * L, L)] = v + N_NODES
            posc[pl.ds(k * L, L)] = win_pos[pl.ds(cb + k * L, L)]
            return 0
        lax.fori_loop(0, CH // L, _st, 0)

    def _issue_gathers(idxc, posc, b0, b1, b2, sg):
        pltpu.async_copy(inc_hbm.at[idxc], b0, sg)
        pltpu.async_copy(sqr_hbm.at[idxc], b1, sg)
        pltpu.async_copy(incret_hbm.at[posc], b2, sg)

    def _wait_gathers(idxc, posc, b0, b1, b2, sg):
        pltpu.make_async_copy(inc_hbm.at[idxc], b0, sg).wait()
        pltpu.make_async_copy(sqr_hbm.at[idxc], b1, sg).wait()
        pltpu.make_async_copy(incret_hbm.at[posc], b2, sg).wait()

    def _compute(b0, b1, b2):
        def _c(q, _):
            r = q // (MEM_DIM // L)
            k = q % (MEM_DIM // L)
            t = b2[r, pl.ds(k * L, L)]
            b0[r, pl.ds(k * L, L)] = b0[r, pl.ds(k * L, L)] + (t + 1.0)
            b1[r, pl.ds(k * L, L)] = b1[r, pl.ds(k * L, L)] + t * t
            return 0
        lax.fori_loop(0, CH * (MEM_DIM // L), _c, 0)

    def _issue_scatters(idxc, idxnc, b0, b1, ss):
        pltpu.async_copy(b0, out_ref.at[idxc], ss)
        pltpu.async_copy(b1, out_ref.at[idxnc], ss)

    def _wait_scatters(idxc, idxnc, b0, b1, ss):
        pltpu.make_async_copy(b0, out_ref.at[idxc], ss).wait()
        pltpu.make_async_copy(b1, out_ref.at[idxnc], ss).wait()

    set_a = (idxchunk, idxnchunk, poschunk, buf0, buf1, buf2, sem_g, sem_s)
    set_b = (idxchunk2, idxnchunk2, poschunk2, buf0b, buf1b, buf2b,
             sem_g2, sem_s2)

    @pl.when(nch > 0)
    def _prolog():
        _stage(0, idxchunk, idxnchunk, poschunk)
        _issue_gathers(idxchunk, poschunk, buf0, buf1, buf2, sem_g)

    def _chunk(ci, _):
        def _iter(cur, nxt):
            ic, inc_, pc, b0, b1, b2, sg, ss = cur
            ic2, inc2, pc2, b0n, b1n, b2n, sg2, ss2 = nxt

            @pl.when(ci + 1 < nch)
            def _prefetch():
                @pl.when(ci >= 1)
                def _drain_prev():
                    _wait_scatters(ic2, inc2, b0n, b1n, ss2)
                _stage(ci + 1, ic2, inc2, pc2)
                _issue_gathers(ic2, pc2, b0n, b1n, b2n, sg2)

            _wait_gathers(ic, pc, b0, b1, b2, sg)
            _compute(b0, b1, b2)
            _issue_scatters(ic, inc_, b0, b1, ss)

        @pl.when(ci % 2 == 0)
        def _even():
            _iter(set_a, set_b)

        @pl.when(ci % 2 == 1)
        def _odd():
            _iter(set_b, set_a)
        return 0
    lax.fori_loop(0, nch, _chunk, 0)

    def _drain(p):
        @pl.when(p % 2 == 0)
        def _a():
            _wait_scatters(idxchunk, idxnchunk, buf0, buf1, sem_s)

        @pl.when(p % 2 == 1)
        def _b():
            _wait_scatters(idxchunk2, idxnchunk2, buf0b, buf1b, sem_s2)

    @pl.when(nch > 0)
    def _ep1():
        _drain(nch - 1)

    @pl.when(nch > 1)
    def _ep2():
        _drain(nch - 2)


@jax.jit
def _sc_call(node_idxs, incret, incretment, incretment_sqr):
    mesh = plsc.VectorSubcoreMesh(core_axis_name="c", subcore_axis_name="s",
                                  num_cores=2, num_subcores=16)
    f = pl.kernel(
        _sc_body,
        out_type=(),
        mesh=mesh,
        compiler_params=pltpu.CompilerParams(needs_layout_passes=False),
        scratch_types=[
            pltpu.VMEM((BATCH,), jnp.int32),
            pltpu.VMEM((BATCH,), jnp.int32),
            pltpu.VMEM((W_PAD,), jnp.int32),
            pltpu.VMEM((BATCH,), jnp.int32),
            pltpu.VMEM((BATCH,), jnp.int32),
            pltpu.VMEM((CH,), jnp.int32),
            pltpu.VMEM((CH,), jnp.int32),
            pltpu.VMEM((CH,), jnp.int32),
            pltpu.VMEM((CH, MEM_DIM), jnp.float32),
            pltpu.VMEM((CH, MEM_DIM), jnp.float32),
            pltpu.VMEM((CH, MEM_DIM), jnp.float32),
            pltpu.VMEM((CH,), jnp.int32),
            pltpu.VMEM((CH,), jnp.int32),
            pltpu.VMEM((CH,), jnp.int32),
            pltpu.VMEM((CH, MEM_DIM), jnp.float32),
            pltpu.VMEM((CH, MEM_DIM), jnp.float32),
            pltpu.VMEM((CH, MEM_DIM), jnp.float32),
            pltpu.SemaphoreType.DMA,
            pltpu.SemaphoreType.DMA,
            pltpu.SemaphoreType.DMA,
            pltpu.SemaphoreType.DMA,
        ],
    )
    out = jax.new_ref(
        jnp.concatenate([incretment, incretment_sqr], axis=0))
    f(node_idxs, incret, incretment, incretment_sqr, out)
    return out[...]


def kernel(node_idxs, incret, incretment, incretment_sqr):
    out = _sc_call(node_idxs.astype(jnp.int32), incret,
                   incretment, incretment_sqr)
    return out.reshape(2, N_NODES, MEM_DIM)


# speedup vs baseline: 1.0931x; 1.0931x over previous
import jax
import jax.numpy as jnp
from jax import lax
from jax.experimental import pallas as pl
from jax.experimental.pallas import tpu as pltpu
from jax.experimental.pallas import tpu_sc as plsc

N_NODES = 100000
MEM_DIM = 128
BATCH = 16384

L = 16
NT = 32
SZ = N_NODES // NT
W_PAD = ((SZ + L - 1) // L) * L
CH = 64


def _sc_body(idx_hbm, incret_hbm, inc_hbm, sqr_hbm, out_ref,
             idx_buf, owned_pos, wtab, win_idx, win_pos,
             idxchunk, idxnchunk, poschunk, buf0, buf1, buf2,
             idxchunk2, idxnchunk2, poschunk2, buf0b, buf1b, buf2b,
             sem_g, sem_s, sem_g2, sem_s2):
    cid = lax.axis_index("c")
    sid = lax.axis_index("s")
    wid = sid * 2 + cid
    base = wid * SZ

    pltpu.sync_copy(idx_hbm, idx_buf)

    def _init(j, _):
        wtab[pl.ds(j * L, L)] = jnp.full((L,), -1, jnp.int32)
        return 0
    lax.fori_loop(0, W_PAD // L, _init, 0)

    def _pass_a(j, cnt):
        v = idx_buf[pl.ds(j * L, L)]
        m = (v >= base) & (v < base + SZ)
        mi = m.astype(jnp.int32)
        s = plsc.cumsum(mi)
        off = cnt + s - 1
        posv = j * L + lax.iota(jnp.int32, L)
        plsc.store_scatter(owned_pos, [off], posv, mask=m)
        return cnt + s[15]
    cnt = lax.fori_loop(0, BATCH // L, _pass_a, jnp.int32(0))

    def _pass_b(j, _):
        valid = j * L + lax.iota(jnp.int32, L) < cnt
        p = owned_pos[pl.ds(j * L, L)]
        v = plsc.load_gather(idx_buf, [p], mask=valid)
        local = v - base
        _, lastm = plsc.scan_count(local, mask=valid)
        plsc.store_scatter(wtab, [local], p, mask=lastm & valid)
        return 0
    lax.fori_loop(0, (cnt + L - 1) // L, _pass_b, 0)

    def _pass_c(j, wcnt):
        w = wtab[pl.ds(j * L, L)]
        m = w >= 0
        mi = m.astype(jnp.int32)
        s = plsc.cumsum(mi)
        off = wcnt + s - 1
        nodev = base + j * L + lax.iota(jnp.int32, L)
        plsc.store_scatter(win_idx, [off], nodev, mask=m)
        plsc.store_scatter(win_pos, [off], w, mask=m)
        return wcnt + s[15]
    wcnt = lax.fori_loop(0, W_PAD // L, _pass_c, jnp.int32(0))

    nch = (wcnt + CH - 1) // CH

    @pl.when(wcnt > 0)
    def _pad():
        def _fill(j, _):
            inds = wcnt + j * L + lax.iota(jnp.int32, L)
            m = inds < nch * CH
            srcs = (inds - wcnt) % wcnt
            nv = plsc.load_gather(win_idx, [srcs], mask=m)
            pv = plsc.load_gather(win_pos, [srcs], mask=m)
            plsc.store_scatter(win_idx, [inds], nv, mask=m)
            plsc.store_scatter(win_pos, [inds], pv, mask=m)
            return 0
        lax.fori_loop(0, CH // L, _fill, 0)

    def _stage(ci, idxc, idxnc, posc):
        cb = ci * CH

        def _st(k, _):
            v = win_idx[pl.ds(cb + k * L, L)]
            idxc[pl.ds(k * L, L)] = v
            idxnc[pl.ds(k * L, L)] = v + N_NODES
            posc[pl.ds(k * L, L)] = win_pos[pl.ds(cb + k * L, L)]
            return 0
        lax.fori_loop(0, CH // L, _st, 0)

    def _issue_gathers(idxc, posc, b0, b1, b2, sg):
        pltpu.async_copy(inc_hbm.at[idxc], b0, sg)
        pltpu.async_copy(sqr_hbm.at[idxc], b1, sg)
        pltpu.async_copy(incret_hbm.at[posc], b2, sg)

    def _wait_gathers(idxc, posc, b0, b1, b2, sg):
        pltpu.make_async_copy(inc_hbm.at[idxc], b0, sg).wait()
        pltpu.make_async_copy(sqr_hbm.at[idxc], b1, sg).wait()
        pltpu.make_async_copy(incret_hbm.at[posc], b2, sg).wait()

    def _compute(b0, b1, b2):
        @plsc.parallel_loop(0, CH * (MEM_DIM // L), unroll=4)
        def _c(q):
            r = q // (MEM_DIM // L)
            k = q % (MEM_DIM // L)
            t = b2[r, pl.ds(k * L, L)]
            b0[r, pl.ds(k * L, L)] = b0[r, pl.ds(k * L, L)] + (t + 1.0)
            b1[r, pl.ds(k * L, L)] = b1[r, pl.ds(k * L, L)] + t * t

    def _issue_scatters(idxc, idxnc, b0, b1, ss):
        pltpu.async_copy(b0, out_ref.at[idxc], ss)
        pltpu.async_copy(b1, out_ref.at[idxnc], ss)

    def _wait_scatters(idxc, idxnc, b0, b1, ss):
        pltpu.make_async_copy(b0, out_ref.at[idxc], ss).wait()
        pltpu.make_async_copy(b1, out_ref.at[idxnc], ss).wait()

    set_a = (idxchunk, idxnchunk, poschunk, buf0, buf1, buf2, sem_g, sem_s)
    set_b = (idxchunk2, idxnchunk2, poschunk2, buf0b, buf1b, buf2b,
             sem_g2, sem_s2)

    @pl.when(nch > 0)
    def _prolog():
        _stage(0, idxchunk, idxnchunk, poschunk)
        _issue_gathers(idxchunk, poschunk, buf0, buf1, buf2, sem_g)

    def _chunk(ci, _):
        def _iter(cur, nxt):
            ic, inc_, pc, b0, b1, b2, sg, ss = cur
            ic2, inc2, pc2, b0n, b1n, b2n, sg2, ss2 = nxt

            @pl.when(ci + 1 < nch)
            def _prefetch():
                @pl.when(ci >= 1)
                def _drain_prev():
                    _wait_scatters(ic2, inc2, b0n, b1n, ss2)
                _stage(ci + 1, ic2, inc2, pc2)
                _issue_gathers(ic2, pc2, b0n, b1n, b2n, sg2)

            _wait_gathers(ic, pc, b0, b1, b2, sg)
            _compute(b0, b1, b2)
            _issue_scatters(ic, inc_, b0, b1, ss)

        @pl.when(ci % 2 == 0)
        def _even():
            _iter(set_a, set_b)

        @pl.when(ci % 2 == 1)
        def _odd():
            _iter(set_b, set_a)
        return 0
    lax.fori_loop(0, nch, _chunk, 0)

    def _drain(p):
        @pl.when(p % 2 == 0)
        def _a():
            _wait_scatters(idxchunk, idxnchunk, buf0, buf1, sem_s)

        @pl.when(p % 2 == 1)
        def _b():
            _wait_scatters(idxchunk2, idxnchunk2, buf0b, buf1b, sem_s2)

    @pl.when(nch > 0)
    def _ep1():
        _drain(nch - 1)

    @pl.when(nch > 1)
    def _ep2():
        _drain(nch - 2)


@jax.jit
def _sc_call(node_idxs, incret, incretment, incretment_sqr):
    mesh = plsc.VectorSubcoreMesh(core_axis_name="c", subcore_axis_name="s",
                                  num_cores=2, num_subcores=16)
    f = pl.kernel(
        _sc_body,
        out_type=(),
        mesh=mesh,
        compiler_params=pltpu.CompilerParams(needs_layout_passes=False),
        scratch_types=[
            pltpu.VMEM((BATCH,), jnp.int32),
            pltpu.VMEM((BATCH,), jnp.int32),
            pltpu.VMEM((W_PAD,), jnp.int32),
            pltpu.VMEM((BATCH,), jnp.int32),
            pltpu.VMEM((BATCH,), jnp.int32),
            pltpu.VMEM((CH,), jnp.int32),
            pltpu.VMEM((CH,), jnp.int32),
            pltpu.VMEM((CH,), jnp.int32),
            pltpu.VMEM((CH, MEM_DIM), jnp.float32),
            pltpu.VMEM((CH, MEM_DIM), jnp.float32),
            pltpu.VMEM((CH, MEM_DIM), jnp.float32),
            pltpu.VMEM((CH,), jnp.int32),
            pltpu.VMEM((CH,), jnp.int32),
            pltpu.VMEM((CH,), jnp.int32),
            pltpu.VMEM((CH, MEM_DIM), jnp.float32),
            pltpu.VMEM((CH, MEM_DIM), jnp.float32),
            pltpu.VMEM((CH, MEM_DIM), jnp.float32),
            pltpu.SemaphoreType.DMA,
            pltpu.SemaphoreType.DMA,
            pltpu.SemaphoreType.DMA,
            pltpu.SemaphoreType.DMA,
        ],
    )
    out = jax.new_ref(
        jnp.concatenate([incretment, incretment_sqr], axis=0))
    f(node_idxs, incret, incretment, incretment_sqr, out)
    return out[...]


def kernel(node_idxs, incret, incretment, incretment_sqr):
    out = _sc_call(node_idxs.astype(jnp.int32), incret,
                   incretment, incretment_sqr)
    return out.reshape(2, N_NODES, MEM_DIM)
